# baseline (speedup 1.0000x reference)
import functools

import jax
import jax.numpy as jnp
from jax import lax
from jax.experimental import pallas as pl
from jax.experimental.pallas import tpu as pltpu
from jax.experimental.pallas import tpu_sc as plsc

_NC = 2
_NS = 16
_NW = _NC * _NS
_INV_SCALE = 1.0 / 30.0
_SQRT_HALF = 0.7071067811865476


def _gelu(x):
    return x * (0.5 * (lax.erf(x * _SQRT_HALF) + 1.0))


def _ln(x, g, b):
    m = jnp.mean(x, axis=-1, keepdims=True)
    d = x - m
    v = jnp.mean(d * d, axis=-1, keepdims=True)
    return d * lax.rsqrt(v + 1e-5) * g + b


def _dotT(x, w):
    return lax.dot_general(x, w, (((1,), (1,)), ((), ())),
                           preferred_element_type=jnp.float32)


def _sc_gather(table, idx3):
    nw, nch, ch = idx3.shape
    v, c = table.shape
    e = nw * nch * ch
    per_w = nch * ch

    mesh = plsc.VectorSubcoreMesh(core_axis_name="c", subcore_axis_name="s")

    @functools.partial(
        pl.kernel,
        out_type=jax.ShapeDtypeStruct((e, c), table.dtype),
        mesh=mesh,
        scratch_types=[
            pltpu.VMEM((nch, ch), jnp.int32),
            pltpu.VMEM((ch, c), table.dtype),
            pltpu.VMEM((ch, c), table.dtype),
            pltpu.SemaphoreType.DMA,
            pltpu.SemaphoreType.DMA,
        ],
    )
    def k(table_hbm, idx_hbm, out_hbm, idx_v, buf0, buf1, sem0, sem1):
        wid = lax.axis_index("s") * _NC + lax.axis_index("c")
        base = wid * per_w
        pltpu.sync_copy(idx_hbm.at[wid], idx_v)
        bufs = (buf0, buf1)
        sems = (sem0, sem1)

        def start(chunk, b):
            pltpu.make_async_copy(
                table_hbm.at[idx_v.at[chunk]], bufs[b], sems[b]
            ).start()

        def wait(b):
            pltpu.make_async_copy(
                table_hbm.at[idx_v.at[0]], bufs[b], sems[b]
            ).wait()

        start(0, 0)

        @pl.when(nch > 1)
        def _():
            start(1, 1)

        @pl.loop(0, (nch + 1) // 2)
        def _(p):
            for b in range(2):
                chunk = p * 2 + b

                @pl.when(chunk < nch)
                def _():
                    wait(b)
                    pltpu.sync_copy(
                        bufs[b], out_hbm.at[pl.ds(base + chunk * ch, ch)])
                    nxt = chunk + 2

                    @pl.when(nxt < nch)
                    def _():
                        start(nxt, b)

    return k(table, idx3)


def _body_a(acc_ref, hv_ref, he_ref, g_ref, ma_ref, mv_ref,
            w1_ref, b1_ref, w2_ref, b2_ref, w3_ref, b3_ref,
            l1g_ref, l1b_ref, win_ref, bin_ref, wout_ref, bout_ref,
            l2g_ref, l2b_ref, out_ref):
    _, t, k, cc = he_ref.shape
    tk = t * k
    hv = hv_ref[0]
    w1 = w1_ref[...]
    he = he_ref[0].reshape(tk, cc)
    pre = _dotT(hv, w1[:, :cc]) + b1_ref[...]
    m = _dotT(he, w1[:, cc:2 * cc]) + _dotT(g_ref[...], w1[:, 2 * cc:])
    x = m.reshape(t, k, cc) + pre[:, None, :]
    x = _gelu(x).reshape(tk, cc)
    x = _gelu(_dotT(x, w2_ref[...]) + b2_ref[...])
    x = _dotT(x, w3_ref[...]) + b3_ref[...]
    x = x.reshape(t, k, cc) * ma_ref[0][:, :, None]
    dh = jnp.sum(x, axis=1) * _INV_SCALE
    h = _ln(hv + dh, l1g_ref[...], l1b_ref[...])
    f = _gelu(_dotT(h, win_ref[...]) + bin_ref[...])
    f = _dotT(f, wout_ref[...]) + bout_ref[...]
    y = _ln(h + f, l2g_ref[...], l2b_ref[...]) * mv_ref[...]
    out_ref[...] = y[None]


def _body_b(acc_ref, hv_ref, he_ref, g_ref,
            w1_ref, b1_ref, w2_ref, b2_ref, w3_ref, b3_ref,
            l3g_ref, l3b_ref, out_ref):
    _, t, k, cc = he_ref.shape
    tk = t * k
    w1 = w1_ref[...]
    he = he_ref[0].reshape(tk, cc)
    pre = _dotT(hv_ref[0], w1[:, :cc]) + b1_ref[...]
    m = _dotT(he, w1[:, cc:2 * cc]) + _dotT(g_ref[...], w1[:, 2 * cc:])
    x = m.reshape(t, k, cc) + pre[:, None, :]
    x = _gelu(x).reshape(tk, cc)
    x = _gelu(_dotT(x, w2_ref[...]) + b2_ref[...])
    x = _dotT(x, w3_ref[...]) + b3_ref[...]
    x = _ln(he + x, l3g_ref[...], l3b_ref[...])
    out_ref[...] = x.reshape(1, t, k, cc)


def kernel(h_V, h_E, E_idx, mask_V, mask_attend,
           W1_w, W1_b, W2_w, W2_b, W3_w, W3_b,
           W11_w, W11_b, W12_w, W12_b, W13_w, W13_b,
           Win_w, Win_b, Wout_w, Wout_b,
           ln1_g, ln1_b, ln2_g, ln2_b, ln3_g, ln3_b):
    bsz, n, k = E_idx.shape
    c = h_V.shape[-1]
    e = n * k
    hv2 = h_V.reshape(n, c)
    mv = mask_V.reshape(n, 1)
    idx = E_idx.reshape(e).astype(jnp.int32)

    ns = 10
    t = 200
    s_nodes = n // ns
    tps = s_nodes // t
    s_edges = s_nodes * k
    tk = t * k
    ch = 40
    per_w = s_edges // _NW
    nch = per_w // ch

    def plan(sizes):
        starts, call_of, s0 = [], {}, 0
        for j, m in enumerate(sizes):
            starts.append(s0)
            for u in range(m):
                call_of[s0 + u] = (j, u)
            s0 += m
        return starts, call_of

    g1_sizes = [1, 1, 2, 3, 3]
    g2_sizes = [1, 2, 3, 4]
    g1_starts, g1_call = plan(g1_sizes)
    g2_starts, g2_call = plan(g2_sizes)

    def idx3(starts, sizes, j):
        a = starts[j] * s_edges
        m = sizes[j]
        return idx[a:a + m * s_edges].reshape(_NW, m * nch, ch)

    def row(x):
        return x.reshape(1, -1)

    wa = (W1_w, row(W1_b), W2_w, row(W2_b), W3_w, row(W3_b),
          row(ln1_g), row(ln1_b), Win_w, row(Win_b), Wout_w, row(Wout_b),
          row(ln2_g), row(ln2_b))
    wb = (W11_w, row(W11_b), W12_w, row(W12_b), W13_w, row(W13_b),
          row(ln3_g), row(ln3_b))

    def full(x):
        return pl.BlockSpec(x.shape, lambda i: tuple(0 for _ in x.shape))

    any_spec = pl.BlockSpec(memory_space=pl.ANY)

    def he_spec(s):
        return pl.BlockSpec((1, t, k, c), lambda i, s=s: (0, s * tps + i, 0, 0))

    def hv_spec(s):
        return pl.BlockSpec((1, t, c), lambda i, s=s: (0, s * tps + i, 0))

    def phase_a(s, acc, g1_s, local):
        specs = [hv_spec(s),
                 he_spec(s),
                 pl.BlockSpec((tk, c), lambda i, u=local: (u * tps + i, 0)),
                 pl.BlockSpec((1, t, k), lambda i, s=s: (0, s * tps + i, 0)),
                 pl.BlockSpec((t, 1), lambda i, s=s: (s * tps + i, 0)),
                 ] + [full(w) for w in wa]
        args = [h_V, h_E, g1_s, mask_attend, mv] + list(wa)
        alias = {}
        if acc is not None:
            specs = [any_spec] + specs
            args = [acc] + args
            alias = {0: 0}
        body = _body_a if acc is not None else (
            lambda *rs: _body_a(None, *rs))
        return pl.pallas_call(
            body,
            grid=(tps,),
            in_specs=specs,
            out_specs=hv_spec(s),
            out_shape=jax.ShapeDtypeStruct((1, n, c), jnp.float32),
            input_output_aliases=alias,
            compiler_params=pltpu.CompilerParams(
                dimension_semantics=("arbitrary",)),
        )(*args)

    def phase_b(s, acc, hv_new, g2_s, local):
        specs = [hv_spec(s),
                 he_spec(s),
                 pl.BlockSpec((tk, c), lambda i, u=local: (u * tps + i, 0)),
                 ] + [full(w) for w in wb]
        args = [hv_new, h_E, g2_s] + list(wb)
        alias = {}
        if acc is not None:
            specs = [any_spec] + specs
            args = [acc] + args
            alias = {0: 0}
        body = _body_b if acc is not None else (
            lambda *rs: _body_b(None, *rs))
        return pl.pallas_call(
            body,
            grid=(tps,),
            in_specs=specs,
            out_specs=he_spec(s),
            out_shape=jax.ShapeDtypeStruct((1, n, k, c), jnp.float32),
            input_output_aliases=alias,
            compiler_params=pltpu.CompilerParams(
                dimension_semantics=("arbitrary",)),
        )(*args)

    g1 = [_sc_gather(hv2, idx3(g1_starts, g1_sizes, j))
          for j in range(len(g1_sizes))]

    acc = None
    for s in range(ns):
        j, local = g1_call[s]
        acc = phase_a(s, acc, g1[j], local)
    hv_new = acc

    g2 = [_sc_gather(hv_new.reshape(n, c), idx3(g2_starts, g2_sizes, j))
          for j in range(len(g2_sizes))]

    acc_e = None
    for s in range(ns):
        j, local = g2_call[s]
        acc_e = phase_b(s, acc_e, hv_new, g2[j], local)

    return hv_new, acc_e


# speedup vs baseline: 1.0095x; 1.0095x over previous
import functools

import jax
import jax.numpy as jnp
from jax import lax
from jax.experimental import pallas as pl
from jax.experimental.pallas import tpu as pltpu
from jax.experimental.pallas import tpu_sc as plsc

_NC = 2
_NS = 16
_NW = _NC * _NS
_INV_SCALE = 1.0 / 30.0
_SQRT_HALF = 0.7071067811865476


def _gelu(x):
    return x * (0.5 * (lax.erf(x * _SQRT_HALF) + 1.0))


def _ln(x, g, b):
    m = jnp.mean(x, axis=-1, keepdims=True)
    d = x - m
    v = jnp.mean(d * d, axis=-1, keepdims=True)
    return d * lax.rsqrt(v + 1e-5) * g + b


def _dotT(x, w):
    return lax.dot_general(x, w, (((1,), (1,)), ((), ())),
                           preferred_element_type=jnp.float32)


def _sc_gather(table, idx_flat, start_edge, e_call, ch):
    v, c = table.shape
    per_w = e_call // _NW
    nch = per_w // ch

    mesh = plsc.VectorSubcoreMesh(core_axis_name="c", subcore_axis_name="s")

    @functools.partial(
        pl.kernel,
        out_type=jax.ShapeDtypeStruct((e_call, c), table.dtype),
        mesh=mesh,
        scratch_types=[
            pltpu.VMEM((per_w,), jnp.int32),
            pltpu.VMEM((ch, c), table.dtype),
            pltpu.VMEM((ch, c), table.dtype),
            pltpu.SemaphoreType.DMA,
            pltpu.SemaphoreType.DMA,
        ],
    )
    def k(table_hbm, idx_hbm, out_hbm, idx_v, buf0, buf1, sem0, sem1):
        wid = lax.axis_index("s") * _NC + lax.axis_index("c")
        base = wid * per_w
        pltpu.sync_copy(idx_hbm.at[pl.ds(start_edge + base, per_w)], idx_v)
        bufs = (buf0, buf1)
        sems = (sem0, sem1)

        def start(chunk, b):
            pltpu.make_async_copy(
                table_hbm.at[idx_v.at[pl.ds(chunk * ch, ch)]],
                bufs[b], sems[b],
            ).start()

        def wait(b):
            pltpu.make_async_copy(
                table_hbm.at[idx_v.at[pl.ds(0, ch)]], bufs[b], sems[b]
            ).wait()

        start(0, 0)

        @pl.when(nch > 1)
        def _():
            start(1, 1)

        @pl.loop(0, (nch + 1) // 2)
        def _(p):
            for b in range(2):
                chunk = p * 2 + b

                @pl.when(chunk < nch)
                def _():
                    wait(b)
                    pltpu.sync_copy(
                        bufs[b], out_hbm.at[pl.ds(base + chunk * ch, ch)])
                    nxt = chunk + 2

                    @pl.when(nxt < nch)
                    def _():
                        start(nxt, b)

    return k(table, idx_flat)


def _body_a(acc_ref, hv_ref, he_ref, g_ref, ma_ref, mv_ref,
            w1_ref, b1_ref, w2_ref, b2_ref, w3_ref, b3_ref,
            l1g_ref, l1b_ref, win_ref, bin_ref, wout_ref, bout_ref,
            l2g_ref, l2b_ref, out_ref):
    _, t, k, cc = he_ref.shape
    tk = t * k
    hv = hv_ref[0]
    w1 = w1_ref[...]
    he = he_ref[0].reshape(tk, cc)
    pre = _dotT(hv, w1[:, :cc]) + b1_ref[...]
    m = _dotT(he, w1[:, cc:2 * cc]) + _dotT(g_ref[...], w1[:, 2 * cc:])
    x = m.reshape(t, k, cc) + pre[:, None, :]
    x = _gelu(x).reshape(tk, cc)
    x = _gelu(_dotT(x, w2_ref[...]) + b2_ref[...])
    x = _dotT(x, w3_ref[...]) + b3_ref[...]
    x = x.reshape(t, k, cc) * ma_ref[0][:, :, None]
    dh = jnp.sum(x, axis=1) * _INV_SCALE
    h = _ln(hv + dh, l1g_ref[...], l1b_ref[...])
    f = _gelu(_dotT(h, win_ref[...]) + bin_ref[...])
    f = _dotT(f, wout_ref[...]) + bout_ref[...]
    y = _ln(h + f, l2g_ref[...], l2b_ref[...]) * mv_ref[...]
    out_ref[...] = y[None]


def _body_b(acc_ref, hv_ref, he_ref, g_ref,
            w1_ref, b1_ref, w2_ref, b2_ref, w3_ref, b3_ref,
            l3g_ref, l3b_ref, out_ref):
    _, t, k, cc = he_ref.shape
    tk = t * k
    w1 = w1_ref[...]
    he = he_ref[0].reshape(tk, cc)
    pre = _dotT(hv_ref[0], w1[:, :cc]) + b1_ref[...]
    m = _dotT(he, w1[:, cc:2 * cc]) + _dotT(g_ref[...], w1[:, 2 * cc:])
    x = m.reshape(t, k, cc) + pre[:, None, :]
    x = _gelu(x).reshape(tk, cc)
    x = _gelu(_dotT(x, w2_ref[...]) + b2_ref[...])
    x = _dotT(x, w3_ref[...]) + b3_ref[...]
    x = _ln(he + x, l3g_ref[...], l3b_ref[...])
    out_ref[...] = x.reshape(1, t, k, cc)


def kernel(h_V, h_E, E_idx, mask_V, mask_attend,
           W1_w, W1_b, W2_w, W2_b, W3_w, W3_b,
           W11_w, W11_b, W12_w, W12_b, W13_w, W13_b,
           Win_w, Win_b, Wout_w, Wout_b,
           ln1_g, ln1_b, ln2_g, ln2_b, ln3_g, ln3_b):
    bsz, n, k = E_idx.shape
    c = h_V.shape[-1]
    e = n * k
    hv2 = h_V.reshape(n, c)
    mv = mask_V.reshape(n, 1)
    idx = E_idx.reshape(e).astype(jnp.int32)

    ns = 10
    t = 200
    s_nodes = n // ns
    tps = s_nodes // t
    s_edges = s_nodes * k
    tk = t * k
    ch = 40
    per_w = s_edges // _NW
    nch = per_w // ch

    def plan(sizes):
        starts, call_of, s0 = [], {}, 0
        for j, m in enumerate(sizes):
            starts.append(s0)
            for u in range(m):
                call_of[s0 + u] = (j, u)
            s0 += m
        return starts, call_of

    g1_sizes = [1, 1, 2, 3, 3]
    g2_sizes = [1, 2, 3, 4]
    g1_starts, g1_call = plan(g1_sizes)
    g2_starts, g2_call = plan(g2_sizes)

    def row(x):
        return x.reshape(1, -1)

    wa = (W1_w, row(W1_b), W2_w, row(W2_b), W3_w, row(W3_b),
          row(ln1_g), row(ln1_b), Win_w, row(Win_b), Wout_w, row(Wout_b),
          row(ln2_g), row(ln2_b))
    wb = (W11_w, row(W11_b), W12_w, row(W12_b), W13_w, row(W13_b),
          row(ln3_g), row(ln3_b))

    def full(x):
        return pl.BlockSpec(x.shape, lambda i: tuple(0 for _ in x.shape))

    any_spec = pl.BlockSpec(memory_space=pl.ANY)

    def he_spec(s):
        return pl.BlockSpec((1, t, k, c), lambda i, s=s: (0, s * tps + i, 0, 0))

    def hv_spec(s):
        return pl.BlockSpec((1, t, c), lambda i, s=s: (0, s * tps + i, 0))

    def phase_a(s, acc, g1_s, local):
        specs = [hv_spec(s),
                 he_spec(s),
                 pl.BlockSpec((tk, c), lambda i, u=local: (u * tps + i, 0)),
                 pl.BlockSpec((1, t, k), lambda i, s=s: (0, s * tps + i, 0)),
                 pl.BlockSpec((t, 1), lambda i, s=s: (s * tps + i, 0)),
                 ] + [full(w) for w in wa]
        args = [h_V, h_E, g1_s, mask_attend, mv] + list(wa)
        alias = {}
        if acc is not None:
            specs = [any_spec] + specs
            args = [acc] + args
            alias = {0: 0}
        body = _body_a if acc is not None else (
            lambda *rs: _body_a(None, *rs))
        return pl.pallas_call(
            body,
            grid=(tps,),
            in_specs=specs,
            out_specs=hv_spec(s),
            out_shape=jax.ShapeDtypeStruct((1, n, c), jnp.float32),
            input_output_aliases=alias,
            compiler_params=pltpu.CompilerParams(
                dimension_semantics=("arbitrary",)),
        )(*args)

    def phase_b(s, acc, hv_new, g2_s, local):
        specs = [hv_spec(s),
                 he_spec(s),
                 pl.BlockSpec((tk, c), lambda i, u=local: (u * tps + i, 0)),
                 ] + [full(w) for w in wb]
        args = [hv_new, h_E, g2_s] + list(wb)
        alias = {}
        if acc is not None:
            specs = [any_spec] + specs
            args = [acc] + args
            alias = {0: 0}
        body = _body_b if acc is not None else (
            lambda *rs: _body_b(None, *rs))
        return pl.pallas_call(
            body,
            grid=(tps,),
            in_specs=specs,
            out_specs=he_spec(s),
            out_shape=jax.ShapeDtypeStruct((1, n, k, c), jnp.float32),
            input_output_aliases=alias,
            compiler_params=pltpu.CompilerParams(
                dimension_semantics=("arbitrary",)),
        )(*args)

    g1 = [_sc_gather(hv2, idx, g1_starts[j] * s_edges,
                     g1_sizes[j] * s_edges, ch)
          for j in range(len(g1_sizes))]

    acc = None
    for s in range(ns):
        j, local = g1_call[s]
        acc = phase_a(s, acc, g1[j], local)
    hv_new = acc

    g2 = [_sc_gather(hv_new.reshape(n, c), idx, g2_starts[j] * s_edges,
                     g2_sizes[j] * s_edges, ch)
          for j in range(len(g2_sizes))]

    acc_e = None
    for s in range(ns):
        j, local = g2_call[s]
        acc_e = phase_b(s, acc_e, hv_new, g2[j], local)

    return hv_new, acc_e


# speedup vs baseline: 1.0160x; 1.0065x over previous
import functools

import jax
import jax.numpy as jnp
from jax import lax
from jax.experimental import pallas as pl
from jax.experimental.pallas import tpu as pltpu
from jax.experimental.pallas import tpu_sc as plsc

_NC = 2
_NS = 16
_NW = _NC * _NS
_INV_SCALE = 1.0 / 30.0
_SQRT_HALF = 0.7071067811865476


def _gelu(x):
    return x * (0.5 * (lax.erf(x * _SQRT_HALF) + 1.0))


def _ln(x, g, b):
    m = jnp.mean(x, axis=-1, keepdims=True)
    d = x - m
    v = jnp.mean(d * d, axis=-1, keepdims=True)
    return d * lax.rsqrt(v + 1e-5) * g + b


def _dotT(x, w):
    return lax.dot_general(x, w, (((1,), (1,)), ((), ())),
                           preferred_element_type=jnp.float32)


def _sc_gather(table, idx_flat, start_edge, e_call, ch):
    v, c = table.shape
    per_w = e_call // _NW
    nch = per_w // ch

    mesh = plsc.VectorSubcoreMesh(core_axis_name="c", subcore_axis_name="s")

    @functools.partial(
        pl.kernel,
        out_type=jax.ShapeDtypeStruct((e_call, c), table.dtype),
        mesh=mesh,
        scratch_types=[
            pltpu.VMEM((per_w,), jnp.int32),
            pltpu.VMEM((ch, c), table.dtype),
            pltpu.VMEM((ch, c), table.dtype),
            pltpu.SemaphoreType.DMA,
            pltpu.SemaphoreType.DMA,
        ],
    )
    def k(table_hbm, idx_hbm, out_hbm, idx_v, buf0, buf1, sem0, sem1):
        wid = lax.axis_index("s") * _NC + lax.axis_index("c")
        base = wid * per_w
        pltpu.sync_copy(idx_hbm.at[pl.ds(start_edge + base, per_w)], idx_v)
        bufs = (buf0, buf1)
        sems = (sem0, sem1)

        def start(chunk, b):
            pltpu.make_async_copy(
                table_hbm.at[idx_v.at[pl.ds(chunk * ch, ch)]],
                bufs[b], sems[b],
            ).start()

        def wait(b):
            pltpu.make_async_copy(
                table_hbm.at[idx_v.at[pl.ds(0, ch)]], bufs[b], sems[b]
            ).wait()

        start(0, 0)

        @pl.when(nch > 1)
        def _():
            start(1, 1)

        @pl.loop(0, (nch + 1) // 2)
        def _(p):
            for b in range(2):
                chunk = p * 2 + b

                @pl.when(chunk < nch)
                def _():
                    wait(b)
                    pltpu.sync_copy(
                        bufs[b], out_hbm.at[pl.ds(base + chunk * ch, ch)])
                    nxt = chunk + 2

                    @pl.when(nxt < nch)
                    def _():
                        start(nxt, b)

    return k(table, idx_flat)


def _body_a(acc_ref, hv_ref, he_ref, g_ref, ma_ref, mv_ref,
            w1_ref, b1_ref, w2_ref, b2_ref, w3_ref, b3_ref,
            l1g_ref, l1b_ref, win_ref, bin_ref, wout_ref, bout_ref,
            l2g_ref, l2b_ref, out_ref):
    _, t, k, cc = he_ref.shape
    tk = t * k
    hv = hv_ref[0]
    w1 = w1_ref[...]
    he = he_ref[0].reshape(tk, cc)
    pre = _dotT(hv, w1[:, :cc]) + b1_ref[...]
    m = _dotT(he, w1[:, cc:2 * cc]) + _dotT(g_ref[...], w1[:, 2 * cc:])
    x = m.reshape(t, k, cc) + pre[:, None, :]
    x = _gelu(x).reshape(tk, cc)
    x = _gelu(_dotT(x, w2_ref[...]) + b2_ref[...])
    x = _dotT(x, w3_ref[...]) + b3_ref[...]
    x = x.reshape(t, k, cc) * ma_ref[0][:, :, None]
    dh = jnp.sum(x, axis=1) * _INV_SCALE
    h = _ln(hv + dh, l1g_ref[...], l1b_ref[...])
    f = _gelu(_dotT(h, win_ref[...]) + bin_ref[...])
    f = _dotT(f, wout_ref[...]) + bout_ref[...]
    y = _ln(h + f, l2g_ref[...], l2b_ref[...]) * mv_ref[...]
    out_ref[...] = y[None]


def _body_b(acc_ref, hv_ref, he_ref, g_ref,
            w1_ref, b1_ref, w2_ref, b2_ref, w3_ref, b3_ref,
            l3g_ref, l3b_ref, out_ref):
    _, t, k, cc = he_ref.shape
    tk = t * k
    w1 = w1_ref[...]
    he = he_ref[0].reshape(tk, cc)
    pre = _dotT(hv_ref[0], w1[:, :cc]) + b1_ref[...]
    m = _dotT(he, w1[:, cc:2 * cc]) + _dotT(g_ref[...], w1[:, 2 * cc:])
    x = m.reshape(t, k, cc) + pre[:, None, :]
    x = _gelu(x).reshape(tk, cc)
    x = _gelu(_dotT(x, w2_ref[...]) + b2_ref[...])
    x = _dotT(x, w3_ref[...]) + b3_ref[...]
    x = _ln(he + x, l3g_ref[...], l3b_ref[...])
    out_ref[...] = x.reshape(1, t, k, cc)


def kernel(h_V, h_E, E_idx, mask_V, mask_attend,
           W1_w, W1_b, W2_w, W2_b, W3_w, W3_b,
           W11_w, W11_b, W12_w, W12_b, W13_w, W13_b,
           Win_w, Win_b, Wout_w, Wout_b,
           ln1_g, ln1_b, ln2_g, ln2_b, ln3_g, ln3_b):
    bsz, n, k = E_idx.shape
    c = h_V.shape[-1]
    e = n * k
    hv2 = h_V.reshape(n, c)
    mv = mask_V.reshape(n, 1)
    idx = E_idx.reshape(e).astype(jnp.int32)

    ns = 10
    t = 200
    s_nodes = n // ns
    tps = s_nodes // t
    s_edges = s_nodes * k
    tk = t * k
    ch = 40
    per_w = s_edges // _NW
    nch = per_w // ch

    def plan(sizes):
        starts, call_of, s0 = [], {}, 0
        for j, m in enumerate(sizes):
            starts.append(s0)
            for u in range(m):
                call_of[s0 + u] = (j, u)
            s0 += m
        return starts, call_of

    g1_sizes = [1, 1, 2, 3, 3]
    g2_sizes = [1, 2, 2, 2, 3]
    g1_starts, g1_call = plan(g1_sizes)
    g2_starts, g2_call = plan(g2_sizes)

    def chunk_rows(m):
        return 40 * m

    def row(x):
        return x.reshape(1, -1)

    wa = (W1_w, row(W1_b), W2_w, row(W2_b), W3_w, row(W3_b),
          row(ln1_g), row(ln1_b), Win_w, row(Win_b), Wout_w, row(Wout_b),
          row(ln2_g), row(ln2_b))
    wb = (W11_w, row(W11_b), W12_w, row(W12_b), W13_w, row(W13_b),
          row(ln3_g), row(ln3_b))

    def full(x):
        return pl.BlockSpec(x.shape, lambda i: tuple(0 for _ in x.shape))

    any_spec = pl.BlockSpec(memory_space=pl.ANY)

    def he_spec(s):
        return pl.BlockSpec((1, t, k, c), lambda i, s=s: (0, s * tps + i, 0, 0))

    def hv_spec(s):
        return pl.BlockSpec((1, t, c), lambda i, s=s: (0, s * tps + i, 0))

    def phase_a(s, acc, g1_s, local):
        specs = [hv_spec(s),
                 he_spec(s),
                 pl.BlockSpec((tk, c), lambda i, u=local: (u * tps + i, 0)),
                 pl.BlockSpec((1, t, k), lambda i, s=s: (0, s * tps + i, 0)),
                 pl.BlockSpec((t, 1), lambda i, s=s: (s * tps + i, 0)),
                 ] + [full(w) for w in wa]
        args = [h_V, h_E, g1_s, mask_attend, mv] + list(wa)
        alias = {}
        if acc is not None:
            specs = [any_spec] + specs
            args = [acc] + args
            alias = {0: 0}
        body = _body_a if acc is not None else (
            lambda *rs: _body_a(None, *rs))
        return pl.pallas_call(
            body,
            grid=(tps,),
            in_specs=specs,
            out_specs=hv_spec(s),
            out_shape=jax.ShapeDtypeStruct((1, n, c), jnp.float32),
            input_output_aliases=alias,
            compiler_params=pltpu.CompilerParams(
                dimension_semantics=("arbitrary",)),
        )(*args)

    def phase_b(s, acc, hv_new, g2_s, local):
        specs = [hv_spec(s),
                 he_spec(s),
                 pl.BlockSpec((tk, c), lambda i, u=local: (u * tps + i, 0)),
                 ] + [full(w) for w in wb]
        args = [hv_new, h_E, g2_s] + list(wb)
        alias = {}
        if acc is not None:
            specs = [any_spec] + specs
            args = [acc] + args
            alias = {0: 0}
        body = _body_b if acc is not None else (
            lambda *rs: _body_b(None, *rs))
        return pl.pallas_call(
            body,
            grid=(tps,),
            in_specs=specs,
            out_specs=he_spec(s),
            out_shape=jax.ShapeDtypeStruct((1, n, k, c), jnp.float32),
            input_output_aliases=alias,
            compiler_params=pltpu.CompilerParams(
                dimension_semantics=("arbitrary",)),
        )(*args)

    g1 = [_sc_gather(hv2, idx, g1_starts[j] * s_edges,
                     g1_sizes[j] * s_edges, chunk_rows(g1_sizes[j]))
          for j in range(len(g1_sizes))]

    acc = None
    for s in range(ns):
        j, local = g1_call[s]
        acc = phase_a(s, acc, g1[j], local)
    hv_new = acc

    g2 = [_sc_gather(hv_new.reshape(n, c), idx, g2_starts[j] * s_edges,
                     g2_sizes[j] * s_edges, chunk_rows(g2_sizes[j]))
          for j in range(len(g2_sizes))]

    acc_e = None
    for s in range(ns):
        j, local = g2_call[s]
        acc_e = phase_b(s, acc_e, hv_new, g2[j], local)

    return hv_new, acc_e


# speedup vs baseline: 1.0904x; 1.0732x over previous
import functools

import jax
import jax.numpy as jnp
from jax import lax
from jax.experimental import pallas as pl
from jax.experimental.pallas import tpu as pltpu
from jax.experimental.pallas import tpu_sc as plsc

_NC = 2
_NS = 16
_NW = _NC * _NS
_INV_SCALE = 1.0 / 30.0
_SQRT_HALF = 0.7071067811865476


def _gelu(x):
    return x * (0.5 * (lax.erf(x * _SQRT_HALF) + 1.0))


def _ln(x, g, b):
    m = jnp.mean(x, axis=-1, keepdims=True)
    d = x - m
    v = jnp.mean(d * d, axis=-1, keepdims=True)
    return d * lax.rsqrt(v + 1e-5) * g + b


def _dotT(x, w):
    return lax.dot_general(x, w, (((1,), (1,)), ((), ())),
                           preferred_element_type=jnp.float32)


def _sc_gather(table, idx_flat, start_edge, e_call, ch):
    v, c = table.shape
    per_w = e_call // _NW
    nch = per_w // ch

    mesh = plsc.VectorSubcoreMesh(core_axis_name="c", subcore_axis_name="s")

    @functools.partial(
        pl.kernel,
        out_type=jax.ShapeDtypeStruct((e_call, c), table.dtype),
        mesh=mesh,
        scratch_types=[
            pltpu.VMEM((per_w,), jnp.int32),
            pltpu.VMEM((ch, c), table.dtype),
            pltpu.VMEM((ch, c), table.dtype),
            pltpu.SemaphoreType.DMA,
            pltpu.SemaphoreType.DMA,
        ],
    )
    def k(table_hbm, idx_hbm, out_hbm, idx_v, buf0, buf1, sem0, sem1):
        wid = lax.axis_index("s") * _NC + lax.axis_index("c")
        base = wid * per_w
        pltpu.sync_copy(idx_hbm.at[pl.ds(start_edge + base, per_w)], idx_v)
        bufs = (buf0, buf1)
        sems = (sem0, sem1)

        def start(chunk, b):
            pltpu.make_async_copy(
                table_hbm.at[idx_v.at[pl.ds(chunk * ch, ch)]],
                bufs[b], sems[b],
            ).start()

        def wait(b):
            pltpu.make_async_copy(
                table_hbm.at[idx_v.at[pl.ds(0, ch)]], bufs[b], sems[b]
            ).wait()

        start(0, 0)

        @pl.when(nch > 1)
        def _():
            start(1, 1)

        @pl.loop(0, (nch + 1) // 2)
        def _(p):
            for b in range(2):
                chunk = p * 2 + b

                @pl.when(chunk < nch)
                def _():
                    wait(b)
                    pltpu.sync_copy(
                        bufs[b], out_hbm.at[pl.ds(base + chunk * ch, ch)])
                    nxt = chunk + 2

                    @pl.when(nxt < nch)
                    def _():
                        start(nxt, b)

    return k(table, idx_flat)


def _body_a(acc_ref, hv_ref, he_ref, g_ref, ma_ref, mv_ref,
            w1_ref, b1_ref, w2_ref, b2_ref, w3_ref, b3_ref,
            l1g_ref, l1b_ref, win_ref, bin_ref, wout_ref, bout_ref,
            l2g_ref, l2b_ref, out_ref):
    _, t, k, cc = he_ref.shape
    tk = t * k
    hv = hv_ref[0]
    w1 = w1_ref[...]
    he = he_ref[0].reshape(tk, cc)
    pre = _dotT(hv, w1[:, :cc]) + b1_ref[...]
    m = _dotT(he, w1[:, cc:2 * cc]) + _dotT(g_ref[...], w1[:, 2 * cc:])
    x = m.reshape(t, k, cc) + pre[:, None, :]
    x = _gelu(x).reshape(tk, cc)
    x = _gelu(_dotT(x, w2_ref[...]) + b2_ref[...])
    x = _dotT(x, w3_ref[...]) + b3_ref[...]
    x = x.reshape(t, k, cc) * ma_ref[0][:, :, None]
    dh = jnp.sum(x, axis=1) * _INV_SCALE
    h = _ln(hv + dh, l1g_ref[...], l1b_ref[...])
    f = _gelu(_dotT(h, win_ref[...]) + bin_ref[...])
    f = _dotT(f, wout_ref[...]) + bout_ref[...]
    y = _ln(h + f, l2g_ref[...], l2b_ref[...]) * mv_ref[...]
    out_ref[...] = y[None]


def _body_b(acc_ref, hv_ref, he_ref, g_ref,
            w1_ref, b1_ref, w2_ref, b2_ref, w3_ref, b3_ref,
            l3g_ref, l3b_ref, out_ref):
    _, t, k, cc = he_ref.shape
    tk = t * k
    w1 = w1_ref[...]
    he = he_ref[0].reshape(tk, cc)
    pre = _dotT(hv_ref[0], w1[:, :cc]) + b1_ref[...]
    m = _dotT(he, w1[:, cc:2 * cc]) + _dotT(g_ref[...], w1[:, 2 * cc:])
    x = m.reshape(t, k, cc) + pre[:, None, :]
    x = _gelu(x).reshape(tk, cc)
    x = _gelu(_dotT(x, w2_ref[...]) + b2_ref[...])
    x = _dotT(x, w3_ref[...]) + b3_ref[...]
    x = _ln(he + x, l3g_ref[...], l3b_ref[...])
    out_ref[...] = x.reshape(1, t, k, cc)


def kernel(h_V, h_E, E_idx, mask_V, mask_attend,
           W1_w, W1_b, W2_w, W2_b, W3_w, W3_b,
           W11_w, W11_b, W12_w, W12_b, W13_w, W13_b,
           Win_w, Win_b, Wout_w, Wout_b,
           ln1_g, ln1_b, ln2_g, ln2_b, ln3_g, ln3_b):
    bsz, n, k = E_idx.shape
    c = h_V.shape[-1]
    e = n * k
    hv2 = h_V.reshape(n, c)
    mv = mask_V.reshape(n, 1)
    idx = E_idx.reshape(e).astype(jnp.int32)

    ns = 10
    t = 200
    s_nodes = n // ns
    tps = s_nodes // t
    s_edges = s_nodes * k
    tk = t * k
    ch = 40
    per_w = s_edges // _NW
    nch = per_w // ch

    def plan(sizes):
        starts, call_of, s0 = [], {}, 0
        for j, m in enumerate(sizes):
            starts.append(s0)
            for u in range(m):
                call_of[s0 + u] = (j, u)
            s0 += m
        return starts, call_of

    g1_sizes = [1, 1, 2, 3, 3]
    g2_sizes = [1, 2, 2, 2, 3]
    g1_starts, g1_call = plan(g1_sizes)
    g2_starts, g2_call = plan(g2_sizes)

    def chunk_rows(m):
        return 40 * m

    def row(x):
        return x.reshape(1, -1)

    wa = (W1_w, row(W1_b), W2_w, row(W2_b), W3_w, row(W3_b),
          row(ln1_g), row(ln1_b), Win_w, row(Win_b), Wout_w, row(Wout_b),
          row(ln2_g), row(ln2_b))
    wb = (W11_w, row(W11_b), W12_w, row(W12_b), W13_w, row(W13_b),
          row(ln3_g), row(ln3_b))

    def full(x):
        return pl.BlockSpec(x.shape, lambda i: tuple(0 for _ in x.shape))

    any_spec = pl.BlockSpec(memory_space=pl.ANY)

    def he_spec(b0):
        return pl.BlockSpec((1, t, k, c), lambda i, b0=b0: (0, b0 + i, 0, 0))

    def hv_spec(b0):
        return pl.BlockSpec((1, t, c), lambda i, b0=b0: (0, b0 + i, 0))

    def phase_a(b0, nblk, acc, g1_s):
        specs = [hv_spec(b0),
                 he_spec(b0),
                 pl.BlockSpec((tk, c), lambda i: (i, 0)),
                 pl.BlockSpec((1, t, k), lambda i, b0=b0: (0, b0 + i, 0)),
                 pl.BlockSpec((t, 1), lambda i, b0=b0: (b0 + i, 0)),
                 ] + [full(w) for w in wa]
        args = [h_V, h_E, g1_s, mask_attend, mv] + list(wa)
        alias = {}
        if acc is not None:
            specs = [any_spec] + specs
            args = [acc] + args
            alias = {0: 0}
        body = _body_a if acc is not None else (
            lambda *rs: _body_a(None, *rs))
        return pl.pallas_call(
            body,
            grid=(nblk,),
            in_specs=specs,
            out_specs=hv_spec(b0),
            out_shape=jax.ShapeDtypeStruct((1, n, c), jnp.float32),
            input_output_aliases=alias,
            compiler_params=pltpu.CompilerParams(
                dimension_semantics=("arbitrary",)),
        )(*args)

    def phase_b(b0, nblk, acc, hv_new, g2_s):
        specs = [hv_spec(b0),
                 he_spec(b0),
                 pl.BlockSpec((tk, c), lambda i: (i, 0)),
                 ] + [full(w) for w in wb]
        args = [hv_new, h_E, g2_s] + list(wb)
        alias = {}
        if acc is not None:
            specs = [any_spec] + specs
            args = [acc] + args
            alias = {0: 0}
        body = _body_b if acc is not None else (
            lambda *rs: _body_b(None, *rs))
        return pl.pallas_call(
            body,
            grid=(nblk,),
            in_specs=specs,
            out_specs=he_spec(b0),
            out_shape=jax.ShapeDtypeStruct((1, n, k, c), jnp.float32),
            input_output_aliases=alias,
            compiler_params=pltpu.CompilerParams(
                dimension_semantics=("arbitrary",)),
        )(*args)

    g1 = [_sc_gather(hv2, idx, g1_starts[j] * s_edges,
                     g1_sizes[j] * s_edges, chunk_rows(g1_sizes[j]))
          for j in range(len(g1_sizes))]

    acc = None
    for j, m in enumerate(g1_sizes):
        acc = phase_a(g1_starts[j] * tps, m * tps, acc, g1[j])
    hv_new = acc

    g2 = [_sc_gather(hv_new.reshape(n, c), idx, g2_starts[j] * s_edges,
                     g2_sizes[j] * s_edges, chunk_rows(g2_sizes[j]))
          for j in range(len(g2_sizes))]

    acc_e = None
    for j, m in enumerate(g2_sizes):
        acc_e = phase_b(g2_starts[j] * tps, m * tps, acc_e, hv_new, g2[j])

    return hv_new, acc_e
